# baseline (speedup 1.0000x reference)
import functools

import jax
import jax.numpy as jnp
from jax import lax
from jax.experimental import pallas as pl
from jax.experimental.pallas import tpu as pltpu
from jax.experimental.pallas import tpu_sc as plsc

B = 16384
DO = 20
NC, NS = 2, 16
NW = NC * NS
BPW = B // NW
NB = BPW // 16


def _lut_body(tab_ref, w21_ref, b21_ref, w22_ref, b22_ref, out_ref):
    h = jnp.maximum(tab_ref[...], 0.0)
    w = jnp.concatenate([w21_ref[...], w22_ref[...]], axis=1)
    lut = jnp.dot(h, w, preferred_element_type=jnp.float32)
    b = jnp.concatenate(
        [b21_ref[...].reshape(1, DO), b22_ref[...].reshape(1, DO)], axis=1)
    out_ref[...] = lut + b


def _make_lut(table, W21, b21, W22, b22):
    return pl.pallas_call(
        _lut_body,
        out_shape=jax.ShapeDtypeStruct((10, 2 * DO), jnp.float32),
    )(table, W21, b21, W22, b22)


@functools.partial(
    pl.kernel,
    out_type=jax.ShapeDtypeStruct((2 * DO, B), jnp.float32),
    mesh=plsc.VectorSubcoreMesh(core_axis_name="c", subcore_axis_name="s"),
    compiler_params=pltpu.CompilerParams(needs_layout_passes=False),
    scratch_types=[
        pltpu.VMEM((BPW,), jnp.int32),
        pltpu.VMEM((10, 2 * DO), jnp.float32),
        pltpu.VMEM((2 * DO, BPW), jnp.float32),
        pltpu.SemaphoreType.DMA,
        pltpu.SemaphoreType.DMA,
        pltpu.SemaphoreType.DMA,
    ],
)
def _sc_gather(x_hbm, lut_hbm, c_hbm, idx_v, lut_v, c_v, sem_i, sem_l, sem_o):
    wid = lax.axis_index("c") * NS + lax.axis_index("s")
    base = wid * BPW
    cp_i = pltpu.async_copy(x_hbm.at[pl.ds(base, BPW)], idx_v, sem_i)
    cp_l = pltpu.async_copy(lut_hbm, lut_v, sem_l)
    cp_i.wait()
    cp_l.wait()

    for bb in range(NB):
        xv = idx_v[pl.ds(bb * 16, 16)]
        for j in range(2 * DO):
            g = plsc.load_gather(lut_v, [xv, jnp.full((16,), j, jnp.int32)])
            c_v[j, pl.ds(bb * 16, 16)] = g
    pltpu.sync_copy(c_v, c_hbm.at[:, pl.ds(base, BPW)])


@jax.jit
def kernel(x, table, W21, b21, W22, b22):
    lut = _make_lut(table, W21, b21, W22, b22)
    c = _sc_gather(x.astype(jnp.int32), lut)
    return c[:DO].T, c[DO:].T


# speedup vs baseline: 1.7925x; 1.7925x over previous
import functools

import jax
import jax.numpy as jnp
from jax import lax
from jax.experimental import pallas as pl
from jax.experimental.pallas import tpu as pltpu
from jax.experimental.pallas import tpu_sc as plsc

B = 16384
DO = 20
NC, NS = 2, 16
NW = NC * NS
BPW = B // NW
NB = BPW // 16


def _lut_body(tab_ref, w21_ref, b21_ref, w22_ref, b22_ref, out_ref):
    h = jnp.maximum(tab_ref[...], 0.0)
    w = jnp.concatenate([w21_ref[...], w22_ref[...]], axis=1)
    lut = jnp.dot(h, w, preferred_element_type=jnp.float32)
    b = jnp.concatenate(
        [b21_ref[...].reshape(1, DO), b22_ref[...].reshape(1, DO)], axis=1)
    lutT = (lut + b).T
    out_ref[...] = jnp.pad(lutT, ((0, 0), (0, 6)))


def _make_lutT(table, W21, b21, W22, b22):
    return pl.pallas_call(
        _lut_body,
        out_shape=jax.ShapeDtypeStruct((2 * DO, 16), jnp.float32),
    )(table, W21, b21, W22, b22)


@functools.partial(
    pl.kernel,
    out_type=(
        jax.ShapeDtypeStruct((DO, B), jnp.float32),
        jax.ShapeDtypeStruct((DO, B), jnp.float32),
    ),
    mesh=plsc.VectorSubcoreMesh(core_axis_name="c", subcore_axis_name="s"),
    compiler_params=pltpu.CompilerParams(needs_layout_passes=False),
    scratch_types=[
        pltpu.VMEM((BPW,), jnp.int32),
        pltpu.VMEM((2 * DO, 16), jnp.float32),
        pltpu.VMEM((DO, BPW), jnp.float32),
        pltpu.VMEM((DO, BPW), jnp.float32),
        pltpu.SemaphoreType.DMA,
        pltpu.SemaphoreType.DMA,
    ],
)
def _sc_gather(x_hbm, lutT_hbm, cmu_hbm, clv_hbm, idx_v, lut_v, cmu_v, clv_v,
               sem_i, sem_l):
    wid = lax.axis_index("c") * NS + lax.axis_index("s")
    base = wid * BPW
    cp_i = pltpu.async_copy(x_hbm.at[pl.ds(base, BPW)], idx_v, sem_i)
    cp_l = pltpu.async_copy(lutT_hbm, lut_v, sem_l)
    cp_i.wait()
    cp_l.wait()
    cols = [lut_v[j, :] for j in range(2 * DO)]
    for bb in range(NB):
        xv = idx_v[pl.ds(bb * 16, 16)]
        for j in range(DO):
            cmu_v[j, pl.ds(bb * 16, 16)] = jnp.take_along_axis(
                cols[j], xv, axis=0)
            clv_v[j, pl.ds(bb * 16, 16)] = jnp.take_along_axis(
                cols[j + DO], xv, axis=0)
    pltpu.sync_copy(cmu_v, cmu_hbm.at[:, pl.ds(base, BPW)])
    pltpu.sync_copy(clv_v, clv_hbm.at[:, pl.ds(base, BPW)])


@jax.jit
def kernel(x, table, W21, b21, W22, b22):
    lutT = _make_lutT(table, W21, b21, W22, b22)
    cmu, clv = _sc_gather(x.astype(jnp.int32), lutT)
    return cmu.T, clv.T


# speedup vs baseline: 1.8725x; 1.0446x over previous
import functools

import jax
import jax.numpy as jnp
from jax import lax
from jax.experimental import pallas as pl
from jax.experimental.pallas import tpu as pltpu
from jax.experimental.pallas import tpu_sc as plsc

B = 16384
DO = 20
NC, NS = 2, 16
NW = NC * NS
BPW = B // NW
NB = BPW // 16


def _lut_body(tab_ref, w21_ref, b21_ref, w22_ref, b22_ref, out_ref):
    h = jnp.maximum(tab_ref[...], 0.0)
    w = jnp.concatenate([w21_ref[...], w22_ref[...]], axis=1)
    lut = jnp.dot(h, w, preferred_element_type=jnp.float32)
    b = jnp.concatenate(
        [b21_ref[...].reshape(1, DO), b22_ref[...].reshape(1, DO)], axis=1)
    lutT = (lut + b).T
    out_ref[...] = jnp.pad(lutT, ((0, 0), (0, 6)))


def _make_lutT(table, W21, b21, W22, b22):
    return pl.pallas_call(
        _lut_body,
        out_shape=jax.ShapeDtypeStruct((2 * DO, 16), jnp.float32),
    )(table, W21, b21, W22, b22)


@functools.partial(
    pl.kernel,
    out_type=(
        jax.ShapeDtypeStruct((DO, B), jnp.float32),
        jax.ShapeDtypeStruct((DO, B), jnp.float32),
    ),
    mesh=plsc.VectorSubcoreMesh(core_axis_name="c", subcore_axis_name="s"),
    compiler_params=pltpu.CompilerParams(needs_layout_passes=False),
    scratch_types=[
        pltpu.VMEM((BPW,), jnp.int32),
        pltpu.VMEM((2 * DO, 16), jnp.float32),
        pltpu.VMEM((DO, BPW), jnp.float32),
        pltpu.VMEM((DO, BPW), jnp.float32),
        pltpu.SemaphoreType.DMA,
        pltpu.SemaphoreType.DMA,
    ],
)
def _sc_gather(x_hbm, lutT_hbm, cmu_hbm, clv_hbm, idx_v, lut_v, cmu_v, clv_v,
               sem_i, sem_l):
    wid = lax.axis_index("c") * NS + lax.axis_index("s")
    base = wid * BPW
    cp_i = pltpu.async_copy(x_hbm.at[pl.ds(base, BPW)], idx_v, sem_i)
    cp_l = pltpu.async_copy(lutT_hbm, lut_v, sem_l)
    cp_i.wait()
    cp_l.wait()
    cols = [lut_v[j, :] for j in range(2 * DO)]

    def body(bb, carry):
        xv = idx_v[pl.ds(bb * 16, 16)]
        for j in range(DO):
            cmu_v[j, pl.ds(bb * 16, 16)] = jnp.take_along_axis(
                cols[j], xv, axis=0)
            clv_v[j, pl.ds(bb * 16, 16)] = jnp.take_along_axis(
                cols[j + DO], xv, axis=0)
        return carry

    lax.fori_loop(0, NB, body, 0)
    pltpu.sync_copy(cmu_v, cmu_hbm.at[:, pl.ds(base, BPW)])
    pltpu.sync_copy(clv_v, clv_hbm.at[:, pl.ds(base, BPW)])


@jax.jit
def kernel(x, table, W21, b21, W22, b22):
    lutT = _make_lutT(table, W21, b21, W22, b22)
    cmu, clv = _sc_gather(x.astype(jnp.int32), lutT)
    return cmu.T, clv.T


# speedup vs baseline: 1.8767x; 1.0023x over previous
import functools

import jax
import jax.numpy as jnp
from jax import lax
from jax.experimental import pallas as pl
from jax.experimental.pallas import tpu as pltpu
from jax.experimental.pallas import tpu_sc as plsc

B = 16384
DO = 20
NC, NS = 2, 16
NW = NC * NS
BPW = B // NW
NB = BPW // 16


def _lut_body(tab_ref, w21_ref, b21_ref, w22_ref, b22_ref, out_ref):
    h = jnp.maximum(tab_ref[...], 0.0)
    w = jnp.concatenate([w21_ref[...], w22_ref[...]], axis=1)
    lut = jnp.dot(h, w, preferred_element_type=jnp.float32)
    b = jnp.concatenate(
        [b21_ref[...].reshape(1, DO), b22_ref[...].reshape(1, DO)], axis=1)
    lutT = (lut + b).T
    out_ref[...] = jnp.pad(lutT, ((0, 0), (0, 6)))


def _make_lutT(table, W21, b21, W22, b22):
    return pl.pallas_call(
        _lut_body,
        out_shape=jax.ShapeDtypeStruct((2 * DO, 16), jnp.float32),
    )(table, W21, b21, W22, b22)


@functools.partial(
    pl.kernel,
    out_type=(
        jax.ShapeDtypeStruct((DO, B), jnp.float32),
        jax.ShapeDtypeStruct((DO, B), jnp.float32),
    ),
    mesh=plsc.VectorSubcoreMesh(core_axis_name="c", subcore_axis_name="s"),
    compiler_params=pltpu.CompilerParams(needs_layout_passes=False),
    scratch_types=[
        pltpu.VMEM((BPW,), jnp.int32),
        pltpu.VMEM((2 * DO, 16), jnp.float32),
        pltpu.VMEM((DO, BPW), jnp.float32),
        pltpu.VMEM((DO, BPW), jnp.float32),
        pltpu.SemaphoreType.DMA,
        pltpu.SemaphoreType.DMA,
    ],
)
def _sc_gather(x_hbm, lutT_hbm, cmu_hbm, clv_hbm, idx_v, lut_v, cmu_v, clv_v,
               sem_i, sem_l):
    wid = lax.axis_index("c") * NS + lax.axis_index("s")
    base = wid * BPW
    cp_i = pltpu.async_copy(x_hbm.at[pl.ds(base, BPW)], idx_v, sem_i)
    cp_l = pltpu.async_copy(lutT_hbm, lut_v, sem_l)
    cp_i.wait()
    cp_l.wait()
    cols = [lut_v[j, :] for j in range(2 * DO)]

    def body(bb, carry):
        xv = idx_v[pl.ds(bb * 16, 16)]
        for j in range(DO):
            cmu_v[j, pl.ds(bb * 16, 16)] = jnp.take_along_axis(
                cols[j], xv, axis=0)
            clv_v[j, pl.ds(bb * 16, 16)] = jnp.take_along_axis(
                cols[j + DO], xv, axis=0)
        return carry

    lax.fori_loop(0, NB, body, 0)
    cp_mu = pltpu.async_copy(cmu_v, cmu_hbm.at[:, pl.ds(base, BPW)], sem_i)
    cp_lv = pltpu.async_copy(clv_v, clv_hbm.at[:, pl.ds(base, BPW)], sem_l)
    cp_mu.wait()
    cp_lv.wait()


@jax.jit
def kernel(x, table, W21, b21, W22, b22):
    lutT = _make_lutT(table, W21, b21, W22, b22)
    cmu, clv = _sc_gather(x.astype(jnp.int32), lutT)
    return cmu.T, clv.T
